# baseline (speedup 1.0000x reference)
import functools

import jax
import jax.numpy as jnp
from jax import lax
from jax.experimental import pallas as pl
from jax.experimental.pallas import tpu as pltpu
from jax.experimental.pallas import tpu_sc as plsc

B, S = 4, 8192
F = 1024
Q = 256
T = 10
N = B * S
NC, NS = 2, 16
NW = NC * NS
RPW = N // NW
CH = 64
NCH = RPW // CH
L = 16
NBUF = 4
LA = 2
SW = 128
TAILW = F - Q - SW


def _body(table, types2d, pos2d, ids2d, hsn2d, lay,
          out,
          gb0, gb1, gb2, gb3, sb0, sb1, sb2, sb3,
          maxbuf, typesv, posi, idsi, posf, specf, hsnf, lay_v, zbuf,
          sg0, sg1, sg2, sg3, sh0, sh1, sh2, sh3,
          ss0, ss1, ss2, ss3, st0, st1, st2, st3):
    wid = lax.axis_index("s") * NC + lax.axis_index("c")
    row0w = wid * RPW
    bidx = row0w // S
    srow0 = row0w % S
    iota = lax.iota(jnp.int32, L)
    z16 = jnp.zeros((L,), jnp.float32)
    gbs = (gb0, gb1, gb2, gb3)
    sbs = (sb0, sb1, sb2, sb3)
    sgs = (sg0, sg1, sg2, sg3)
    shs = (sh0, sh1, sh2, sh3)
    sss = (ss0, ss1, ss2, ss3)
    sts = (st0, st1, st2, st3)

    acc = jnp.full((L,), -1.0, jnp.float32)
    for bb in range(B):
        pltpu.sync_copy(hsn2d.at[bb], maxbuf)

        def _mx(i, a):
            return jnp.maximum(a, maxbuf[pl.ds(i * L, L)])

        acc = lax.fori_loop(0, S // L, _mx, acc)
    maxv = jnp.max(acc)

    def _z0(r, carry):
        for k in range(0, Q, L):
            gb0[r, pl.ds(k, L)] = z16
        return carry

    lax.fori_loop(0, CH, _z0, 0)
    pltpu.sync_copy(gb0, zbuf.at[pl.ds(0, CH), pl.ds(0, Q)])
    pltpu.sync_copy(gb0, zbuf.at[pl.ds(0, CH), pl.ds(Q, Q)])
    pltpu.sync_copy(gb0.at[pl.ds(0, CH), pl.ds(0, TAILW - 2 * Q)],
                    zbuf.at[pl.ds(0, CH), pl.ds(2 * Q, TAILW - 2 * Q)])
    plsc.subcore_barrier()

    pltpu.sync_copy(types2d.at[bidx].at[pl.ds(srow0, RPW)], typesv)
    pltpu.sync_copy(pos2d.at[bidx].at[pl.ds(srow0, RPW)], posi)
    pltpu.sync_copy(ids2d.at[bidx].at[pl.ds(srow0, RPW)], idsi)
    pltpu.sync_copy(hsn2d.at[bidx].at[pl.ds(srow0, RPW)], hsnf)
    pltpu.sync_copy(lay, lay_v)
    layv = lay_v[...]

    def _cols(j, carry):
        sl = pl.ds(j * L, L)
        posf[sl] = posi[sl].astype(jnp.float32) * (1.0 / S)
        specf[sl] = jnp.where(idsi[sl] < 5, 1.0, 0.0).astype(jnp.float32)
        hsnf[sl] = hsnf[sl] / maxv
        return carry

    lax.fori_loop(0, RPW // L, _cols, 0)

    def _zrow(r, carry):
        for sb in sbs:
            for k in range(0, SW, L):
                sb[r, pl.ds(k, L)] = z16
        return carry

    lax.fori_loop(0, CH, _zrow, 0)
    colL = jnp.full((L,), 3, jnp.int32)
    for sb in sbs:
        for g in range(CH // L):
            rows = g * L + iota
            plsc.store_scatter(sb, [rows, colL], layv)

    col0 = jnp.full((L,), 0, jnp.int32)
    col1 = jnp.full((L,), 1, jnp.int32)
    col2 = jnp.full((L,), 2, jnp.int32)
    gth = [None] * NBUF
    out_h = [None] * NBUF
    out_s = [None] * NBUF
    out_t = [None] * NBUF

    for c in range(LA):
        gth[c] = pltpu.async_copy(
            table.at[typesv.at[pl.ds(c * CH, CH)]], gbs[c], sgs[c]
        )

    for c in range(NCH):
        b = c % NBUF
        rows_sl = pl.ds(srow0 + c * CH, CH)

        if out_t[b] is not None:
            out_t[b].wait()
        out_t[b] = pltpu.async_copy(
            zbuf, out.at[bidx, rows_sl, pl.ds(Q + SW, TAILW)], sts[b]
        )

        if out_s[b] is not None:
            out_s[b].wait()
        sb = sbs[b]
        for g in range(CH // L):
            off = c * CH + g * L
            rows = g * L + iota
            plsc.store_scatter(sb, [rows, col0], posf[pl.ds(off, L)])
            plsc.store_scatter(sb, [rows, col1], specf[pl.ds(off, L)])
            plsc.store_scatter(sb, [rows, col2], hsnf[pl.ds(off, L)])
        out_s[b] = pltpu.async_copy(
            sb, out.at[bidx, rows_sl, pl.ds(Q, SW)], sss[b]
        )

        gth[b].wait()
        gth[b] = None
        out_h[b] = pltpu.async_copy(
            gbs[b], out.at[bidx, rows_sl, pl.ds(0, Q)], shs[b]
        )

        cn = c + LA
        if cn < NCH:
            bn = cn % NBUF
            if out_h[bn] is not None:
                out_h[bn].wait()
                out_h[bn] = None
            gth[bn] = pltpu.async_copy(
                table.at[typesv.at[pl.ds(cn * CH, CH)]], gbs[bn], sgs[bn]
            )

    for b in range(NBUF):
        if out_h[b] is not None:
            out_h[b].wait()
        if out_s[b] is not None:
            out_s[b].wait()
        if out_t[b] is not None:
            out_t[b].wait()


@jax.jit
def _run(table, types2d, pos2d, ids2d, hsn2d, lay):
    mesh = plsc.VectorSubcoreMesh(
        core_axis_name="c", subcore_axis_name="s", num_cores=NC, num_subcores=NS
    )
    f = functools.partial(
        pl.kernel,
        out_type=jax.ShapeDtypeStruct((B, S, F), jnp.float32),
        mesh=mesh,
        scratch_types=[
            pltpu.VMEM((CH, Q), jnp.float32),
            pltpu.VMEM((CH, Q), jnp.float32),
            pltpu.VMEM((CH, Q), jnp.float32),
            pltpu.VMEM((CH, Q), jnp.float32),
            pltpu.VMEM((CH, SW), jnp.float32),
            pltpu.VMEM((CH, SW), jnp.float32),
            pltpu.VMEM((CH, SW), jnp.float32),
            pltpu.VMEM((CH, SW), jnp.float32),
            pltpu.VMEM((S,), jnp.float32),
            pltpu.VMEM((RPW,), jnp.int32),
            pltpu.VMEM((RPW,), jnp.int32),
            pltpu.VMEM((RPW,), jnp.int32),
            pltpu.VMEM((RPW,), jnp.float32),
            pltpu.VMEM((RPW,), jnp.float32),
            pltpu.VMEM((RPW,), jnp.float32),
            pltpu.VMEM((L,), jnp.float32),
            pltpu.VMEM_SHARED((CH, TAILW), jnp.float32),
            pltpu.SemaphoreType.DMA,
            pltpu.SemaphoreType.DMA,
            pltpu.SemaphoreType.DMA,
            pltpu.SemaphoreType.DMA,
            pltpu.SemaphoreType.DMA,
            pltpu.SemaphoreType.DMA,
            pltpu.SemaphoreType.DMA,
            pltpu.SemaphoreType.DMA,
            pltpu.SemaphoreType.DMA,
            pltpu.SemaphoreType.DMA,
            pltpu.SemaphoreType.DMA,
            pltpu.SemaphoreType.DMA,
            pltpu.SemaphoreType.DMA,
            pltpu.SemaphoreType.DMA,
            pltpu.SemaphoreType.DMA,
            pltpu.SemaphoreType.DMA,
        ],
        compiler_params=pltpu.CompilerParams(
            use_tc_tiling_on_sc=True, needs_layout_passes=False
        ),
    )(_body)
    return f(table, types2d, pos2d, ids2d, hsn2d, lay)


def kernel(input_ids, token_type_ids, positions, hidden_state_norms,
           layer_idx, token_type_table):
    lay = jnp.zeros((L,), jnp.float32) + jnp.asarray(layer_idx, jnp.float32) / 100.0
    return _run(token_type_table, token_type_ids, positions, input_ids,
                hidden_state_norms, lay)


# speedup vs baseline: 2.1833x; 2.1833x over previous
import functools

import jax
import jax.numpy as jnp
from jax import lax
from jax.experimental import pallas as pl
from jax.experimental.pallas import tpu as pltpu
from jax.experimental.pallas import tpu_sc as plsc

B, S = 4, 8192
F = 1024
Q = 256
T = 10
N = B * S
NC, NS = 2, 16
NW = NC * NS
RPW = N // NW
CH = 32
NCH = RPW // CH
L = 16
NBUF = 3


def _body(table, types2d, pos2d, ids2d, hsn2d, lay,
          out,
          stg0, stg1, stg2, maxbuf, tloc, typesv, posi, idsi, posf,
          specf, hsnf, lay_v,
          sh0, sh1, sh2):
    wid = lax.axis_index("s") * NC + lax.axis_index("c")
    row0w = wid * RPW
    bidx = row0w // S
    srow0 = row0w % S
    iota = lax.iota(jnp.int32, L)
    z16 = jnp.zeros((L,), jnp.float32)
    stgs = (stg0, stg1, stg2)
    shs = (sh0, sh1, sh2)

    acc = jnp.full((L,), -1.0, jnp.float32)
    for bb in range(B):
        pltpu.sync_copy(hsn2d.at[bb], maxbuf)

        def _mx(i, a):
            return jnp.maximum(a, maxbuf[pl.ds(i * L, L)])

        acc = lax.fori_loop(0, S // L, _mx, acc)
    maxv = jnp.max(acc)

    pltpu.sync_copy(table, tloc)
    pltpu.sync_copy(types2d.at[bidx].at[pl.ds(srow0, RPW)], typesv)
    pltpu.sync_copy(pos2d.at[bidx].at[pl.ds(srow0, RPW)], posi)
    pltpu.sync_copy(ids2d.at[bidx].at[pl.ds(srow0, RPW)], idsi)
    pltpu.sync_copy(hsn2d.at[bidx].at[pl.ds(srow0, RPW)], hsnf)
    pltpu.sync_copy(lay, lay_v)
    layv = lay_v[...]

    def _cols(j, carry):
        sl = pl.ds(j * L, L)
        posf[sl] = posi[sl].astype(jnp.float32) * (1.0 / S)
        specf[sl] = jnp.where(idsi[sl] < 5, 1.0, 0.0).astype(jnp.float32)
        hsnf[sl] = hsnf[sl] / maxv
        return carry

    lax.fori_loop(0, RPW // L, _cols, 0)

    def _zrow(r, carry):
        for stg in stgs:
            for k in range(0, F, L):
                stg[r, pl.ds(k, L)] = z16
        return carry

    lax.fori_loop(0, CH, _zrow, 0)
    colL = jnp.full((L,), Q + 3, jnp.int32)
    for stg in stgs:
        for g in range(CH // L):
            rows = g * L + iota
            plsc.store_scatter(stg, [rows, colL], layv)

    cols = [iota + k * L for k in range(Q // L)]
    col0 = jnp.full((L,), Q + 0, jnp.int32)
    col1 = jnp.full((L,), Q + 1, jnp.int32)
    col2 = jnp.full((L,), Q + 2, jnp.int32)
    out_h = [None] * NBUF

    for c in range(NCH):
        b = c % NBUF
        stg = stgs[b]
        rows_sl = pl.ds(srow0 + c * CH, CH)

        if out_h[b] is not None:
            out_h[b].wait()

        def _erow(r, carry, stg=stg, c=c):
            tfull = plsc.load_gather(
                typesv, [jnp.zeros((L,), jnp.int32) + (c * CH + r)])
            for k in range(Q // L):
                stg[r, pl.ds(k * L, L)] = plsc.load_gather(tloc, [tfull, cols[k]])
            return carry

        lax.fori_loop(0, CH, _erow, 0)

        for g in range(CH // L):
            off = c * CH + g * L
            rows = g * L + iota
            plsc.store_scatter(stg, [rows, col0], posf[pl.ds(off, L)])
            plsc.store_scatter(stg, [rows, col1], specf[pl.ds(off, L)])
            plsc.store_scatter(stg, [rows, col2], hsnf[pl.ds(off, L)])

        out_h[b] = pltpu.async_copy(
            stg, out.at[bidx, rows_sl, pl.ds(0, F)], shs[b]
        )

    for b in range(NBUF):
        if out_h[b] is not None:
            out_h[b].wait()


@jax.jit
def _run(table, types2d, pos2d, ids2d, hsn2d, lay):
    mesh = plsc.VectorSubcoreMesh(
        core_axis_name="c", subcore_axis_name="s", num_cores=NC, num_subcores=NS
    )
    f = functools.partial(
        pl.kernel,
        out_type=jax.ShapeDtypeStruct((B, S, F), jnp.float32),
        mesh=mesh,
        scratch_types=[
            pltpu.VMEM((CH, F), jnp.float32),
            pltpu.VMEM((CH, F), jnp.float32),
            pltpu.VMEM((CH, F), jnp.float32),
            pltpu.VMEM((S,), jnp.float32),
            pltpu.VMEM((T, Q), jnp.float32),
            pltpu.VMEM((RPW,), jnp.int32),
            pltpu.VMEM((RPW,), jnp.int32),
            pltpu.VMEM((RPW,), jnp.int32),
            pltpu.VMEM((RPW,), jnp.float32),
            pltpu.VMEM((RPW,), jnp.float32),
            pltpu.VMEM((RPW,), jnp.float32),
            pltpu.VMEM((L,), jnp.float32),
            pltpu.SemaphoreType.DMA,
            pltpu.SemaphoreType.DMA,
            pltpu.SemaphoreType.DMA,
        ],
        compiler_params=pltpu.CompilerParams(
            use_tc_tiling_on_sc=True, needs_layout_passes=False
        ),
    )(_body)
    return f(table, types2d, pos2d, ids2d, hsn2d, lay)


def kernel(input_ids, token_type_ids, positions, hidden_state_norms,
           layer_idx, token_type_table):
    lay = jnp.zeros((L,), jnp.float32) + jnp.asarray(layer_idx, jnp.float32) / 100.0
    return _run(token_type_table, token_type_ids, positions, input_ids,
                hidden_state_norms, lay)


# speedup vs baseline: 2.3067x; 1.0565x over previous
import functools

import jax
import jax.numpy as jnp
from jax import lax
from jax.experimental import pallas as pl
from jax.experimental.pallas import tpu as pltpu
from jax.experimental.pallas import tpu_sc as plsc

B, S = 4, 8192
F = 1024
Q = 256
T = 10
N = B * S
NC, NS = 2, 16
NW = NC * NS
RPW = N // NW
CH = 64
NCH = RPW // CH
L = 16
NBUF = 3
HEAD = 384
TAILW = F - HEAD


def _body(table, types2d, pos2d, ids2d, hsn2d, lay,
          out,
          stg0, stg1, stg2, maxbuf, tloc, typesv, posi, idsi, posf,
          specf, hsnf, lay_v, zbuf,
          sh0, sh1, sh2, st0, st1, st2):
    wid = lax.axis_index("s") * NC + lax.axis_index("c")
    row0w = wid * RPW
    bidx = row0w // S
    srow0 = row0w % S
    iota = lax.iota(jnp.int32, L)
    z16 = jnp.zeros((L,), jnp.float32)
    stgs = (stg0, stg1, stg2)
    shs = (sh0, sh1, sh2)
    sts = (st0, st1, st2)

    pltpu.sync_copy(hsn2d, maxbuf)

    acc = jnp.full((L,), -1.0, jnp.float32)
    for bb in range(B):
        def _mx(i, a, bb=bb):
            return jnp.maximum(a, maxbuf[bb, pl.ds(i * L, L)])

        acc = lax.fori_loop(0, S // L, _mx, acc)
    maxv = jnp.max(acc)

    def _z0(r, carry):
        for k in range(0, HEAD, L):
            stg0[r, pl.ds(k, L)] = z16
        return carry

    lax.fori_loop(0, CH, _z0, 0)
    pltpu.sync_copy(stg0, zbuf.at[pl.ds(0, CH), pl.ds(0, HEAD)])
    pltpu.sync_copy(stg0.at[pl.ds(0, CH), pl.ds(0, TAILW - HEAD)],
                    zbuf.at[pl.ds(0, CH), pl.ds(HEAD, TAILW - HEAD)])
    plsc.subcore_barrier()

    pltpu.sync_copy(table, tloc)
    pltpu.sync_copy(types2d.at[bidx].at[pl.ds(srow0, RPW)], typesv)
    pltpu.sync_copy(pos2d.at[bidx].at[pl.ds(srow0, RPW)], posi)
    pltpu.sync_copy(ids2d.at[bidx].at[pl.ds(srow0, RPW)], idsi)
    pltpu.sync_copy(hsn2d.at[bidx].at[pl.ds(srow0, RPW)], hsnf)
    pltpu.sync_copy(lay, lay_v)
    layv = lay_v[...]

    def _cols(j, carry):
        sl = pl.ds(j * L, L)
        posf[sl] = posi[sl].astype(jnp.float32) * (1.0 / S)
        specf[sl] = jnp.where(idsi[sl] < 5, 1.0, 0.0).astype(jnp.float32)
        hsnf[sl] = hsnf[sl] / maxv
        return carry

    lax.fori_loop(0, RPW // L, _cols, 0)

    def _zrow(r, carry):
        for stg in (stg1, stg2):
            for k in range(Q, HEAD, L):
                stg[r, pl.ds(k, L)] = z16
        return carry

    lax.fori_loop(0, CH, _zrow, 0)
    colL = jnp.full((L,), Q + 3, jnp.int32)
    for stg in stgs:
        for g in range(CH // L):
            rows = g * L + iota
            plsc.store_scatter(stg, [rows, colL], layv)

    cols = [iota + k * L for k in range(Q // L)]
    col0 = jnp.full((L,), Q + 0, jnp.int32)
    col1 = jnp.full((L,), Q + 1, jnp.int32)
    col2 = jnp.full((L,), Q + 2, jnp.int32)
    out_h = [None] * NBUF
    out_t = [None] * NBUF

    for c in range(NCH):
        b = c % NBUF
        stg = stgs[b]
        rows_sl = pl.ds(srow0 + c * CH, CH)

        if out_t[b] is not None:
            out_t[b].wait()
        out_t[b] = pltpu.async_copy(
            zbuf, out.at[bidx, rows_sl, pl.ds(HEAD, TAILW)], sts[b]
        )

        if out_h[b] is not None:
            out_h[b].wait()

        def _erow(i, carry, stg=stg, c=c):
            r = i * 2
            tfull0 = plsc.load_gather(
                typesv, [jnp.zeros((L,), jnp.int32) + (c * CH + r)])
            tfull1 = plsc.load_gather(
                typesv, [jnp.zeros((L,), jnp.int32) + (c * CH + r + 1)])
            for k in range(Q // L):
                stg[r, pl.ds(k * L, L)] = plsc.load_gather(tloc, [tfull0, cols[k]])
            for k in range(Q // L):
                stg[r + 1, pl.ds(k * L, L)] = plsc.load_gather(tloc, [tfull1, cols[k]])
            return carry

        lax.fori_loop(0, CH // 2, _erow, 0)

        for g in range(CH // L):
            off = c * CH + g * L
            rows = g * L + iota
            plsc.store_scatter(stg, [rows, col0], posf[pl.ds(off, L)])
            plsc.store_scatter(stg, [rows, col1], specf[pl.ds(off, L)])
            plsc.store_scatter(stg, [rows, col2], hsnf[pl.ds(off, L)])

        out_h[b] = pltpu.async_copy(
            stg, out.at[bidx, rows_sl, pl.ds(0, HEAD)], shs[b]
        )

    for b in range(NBUF):
        if out_h[b] is not None:
            out_h[b].wait()
        if out_t[b] is not None:
            out_t[b].wait()


@jax.jit
def _run(table, types2d, pos2d, ids2d, hsn2d, lay):
    mesh = plsc.VectorSubcoreMesh(
        core_axis_name="c", subcore_axis_name="s", num_cores=NC, num_subcores=NS
    )
    f = functools.partial(
        pl.kernel,
        out_type=jax.ShapeDtypeStruct((B, S, F), jnp.float32),
        mesh=mesh,
        scratch_types=[
            pltpu.VMEM((CH, HEAD), jnp.float32),
            pltpu.VMEM((CH, HEAD), jnp.float32),
            pltpu.VMEM((CH, HEAD), jnp.float32),
            pltpu.VMEM((B, S), jnp.float32),
            pltpu.VMEM((T, Q), jnp.float32),
            pltpu.VMEM((RPW,), jnp.int32),
            pltpu.VMEM((RPW,), jnp.int32),
            pltpu.VMEM((RPW,), jnp.int32),
            pltpu.VMEM((RPW,), jnp.float32),
            pltpu.VMEM((RPW,), jnp.float32),
            pltpu.VMEM((RPW,), jnp.float32),
            pltpu.VMEM((L,), jnp.float32),
            pltpu.VMEM_SHARED((CH, TAILW), jnp.float32),
            pltpu.SemaphoreType.DMA,
            pltpu.SemaphoreType.DMA,
            pltpu.SemaphoreType.DMA,
            pltpu.SemaphoreType.DMA,
            pltpu.SemaphoreType.DMA,
            pltpu.SemaphoreType.DMA,
        ],
        compiler_params=pltpu.CompilerParams(
            use_tc_tiling_on_sc=True, needs_layout_passes=False
        ),
    )(_body)
    return f(table, types2d, pos2d, ids2d, hsn2d, lay)


def kernel(input_ids, token_type_ids, positions, hidden_state_norms,
           layer_idx, token_type_table):
    lay = jnp.zeros((L,), jnp.float32) + jnp.asarray(layer_idx, jnp.float32) / 100.0
    return _run(token_type_table, token_type_ids, positions, input_ids,
                hidden_state_norms, lay)
